# baseline (speedup 1.0000x reference)
import functools

import jax
import jax.numpy as jnp
from jax import lax
from jax.experimental import pallas as pl
from jax.experimental.pallas import tpu as pltpu
from jax.experimental.pallas import tpu_sc as plsc


def _mm_kernel(x_ref, w_ref, b_ref, y_ref):
    y = jax.lax.dot_general(
        x_ref[...], w_ref[...], (((1,), (1,)), ((), ())),
        preferred_element_type=jnp.float32)
    y_ref[...] = jnp.maximum(y + b_ref[...], 0.0)


def kernel(X, sample_domain, W, b):
    n, d = X.shape
    Y = pl.pallas_call(
        _mm_kernel,
        out_shape=jax.ShapeDtypeStruct((n, d), jnp.float32),
    )(X, W, b.reshape(1, d))

    mesh = plsc.VectorSubcoreMesh(core_axis_name="c", subcore_axis_name="s")
    info = plsc.get_sparse_core_info()
    nsub = info.num_subcores
    nw = info.num_cores * nsub
    rows_per_w = n // nw

    def _bcast_body(y_hbm, o_hbm, spmem):
        c = lax.axis_index("c")
        s = lax.axis_index("s")

        @pl.when(s == 0)
        def _():
            pltpu.sync_copy(y_hbm, spmem)

        plsc.subcore_barrier()
        base = (c * nsub + s) * rows_per_w
        for j in range(rows_per_w):
            pltpu.sync_copy(spmem, o_hbm.at[base + j])

    out = functools.partial(
        pl.kernel,
        out_type=jax.ShapeDtypeStruct((n, n, d), jnp.float32),
        mesh=mesh,
        scratch_types=[pltpu.VMEM_SHARED((n, d), jnp.float32)],
    )(_bcast_body)(Y)
    return out


# speedup vs baseline: 1.9382x; 1.9382x over previous
import jax
import jax.numpy as jnp
from jax.experimental import pallas as pl
from jax.experimental.pallas import tpu as pltpu

_CHUNK = 8


def _dma_kernel(x_ref, w_ref, b_ref, o_ref, y_ref, sem0, sem1):
    y = jax.lax.dot_general(
        x_ref[...], w_ref[...], (((1,), (1,)), ((), ())),
        preferred_element_type=jnp.float32)
    y_ref[...] = jnp.maximum(y + b_ref[...], 0.0)
    n = x_ref.shape[0]
    sems = (sem0, sem1)

    def issue(i, k):
        return pltpu.make_async_copy(y_ref, o_ref.at[i], sems[k])

    for j in range(_CHUNK):
        issue(j, j % 2).start()

    def body(g, _):
        i = g * 2
        issue(i + _CHUNK, 0).start()
        issue(i, 0).wait()
        issue(i + 1 + _CHUNK, 1).start()
        issue(i + 1, 1).wait()
        return 0

    jax.lax.fori_loop(0, (n - _CHUNK) // 2, body, 0)
    for j in range(_CHUNK):
        issue(n - _CHUNK + j, j % 2).wait()


def kernel(X, sample_domain, W, b):
    n, d = X.shape
    out = pl.pallas_call(
        _dma_kernel,
        in_specs=[
            pl.BlockSpec(memory_space=pltpu.VMEM),
            pl.BlockSpec(memory_space=pltpu.VMEM),
            pl.BlockSpec(memory_space=pltpu.VMEM),
        ],
        out_specs=pl.BlockSpec(memory_space=pl.ANY),
        out_shape=jax.ShapeDtypeStruct((n, n, d), jnp.float32),
        scratch_shapes=[
            pltpu.VMEM((n, d), jnp.float32),
            pltpu.SemaphoreType.DMA,
            pltpu.SemaphoreType.DMA,
        ],
    )(X, W, b.reshape(1, d))
    return out
